# baseline (speedup 1.0000x reference)
import functools

import jax
import jax.numpy as jnp
from jax import lax
from jax.experimental import pallas as pl
from jax.experimental.pallas import tpu as pltpu
from jax.experimental.pallas import tpu_sc as plsc

N_NODES = 10000
N_EDGES = 320000
D_FEAT = 128
HIDDEN = 64
N_CLASSES = 2
N_GRAPHS = 16

NC = 2
NS = 16
NW = NC * NS

NACC = 10240
CH = 128
EPW = 10240
EPAD = EPW * NW
CHUNKS = EPW // CH
RPW = N_NODES // NS

_SC_MESH = plsc.VectorSubcoreMesh(
    core_axis_name="c", subcore_axis_name="s", num_cores=NC, num_subcores=NS)


NBUF = 8
GROUPS = CHUNKS // NBUF


def _agg_body(y_ref, src_ref, dst_ref, zero_ref, out_ref, *scratch):
  sidx, didx = scratch[0], scratch[1]
  rows = scratch[2:2 + NBUF]
  gsem = scratch[2 + NBUF:2 + 2 * NBUF]
  ssem = scratch[2 + 2 * NBUF:2 + 3 * NBUF]
  acc = scratch[2 + 3 * NBUF]
  c = lax.axis_index("c")
  s = lax.axis_index("s")
  w = c * NS + s
  rbase = s * RPW
  pltpu.sync_copy(src_ref.at[w], sidx)
  pltpu.sync_copy(dst_ref.at[w], didx)
  pltpu.sync_copy(zero_ref.at[pl.ds(rbase, RPW)], acc.at[pl.ds(rbase, RPW)])
  plsc.subcore_barrier()

  for b in range(NBUF):
    pltpu.async_copy(y_ref.at[sidx.at[b]], rows[b], gsem[b])

  def group(j, carry):
    base = j * NBUF
    for b in range(NBUF):
      pltpu.make_async_copy(y_ref.at[sidx.at[base + b]], rows[b],
                            gsem[b]).wait()
      pltpu.async_copy(rows[b], acc.at[didx.at[base + b]], ssem[b], add=True)
    for b in range(NBUF):
      nxt = jnp.minimum(base + NBUF + b, CHUNKS - 1)
      pltpu.make_async_copy(rows[b], acc.at[didx.at[nxt]], ssem[b]).wait()

      @pl.when(j < GROUPS - 1)
      def _():
        pltpu.async_copy(y_ref.at[sidx.at[nxt]], rows[b], gsem[b])
    return carry

  lax.fori_loop(0, GROUPS, group, 0)
  plsc.subcore_barrier()
  pltpu.sync_copy(acc.at[pl.ds(rbase, RPW)], out_ref.at[c, pl.ds(rbase, RPW)])


_sc_aggregate = pl.kernel(
    _agg_body,
    out_type=jax.ShapeDtypeStruct((NC, N_NODES, HIDDEN), jnp.float32),
    mesh=_SC_MESH,
    scratch_types=[
        pltpu.VMEM((CHUNKS, CH), jnp.int32),
        pltpu.VMEM((CHUNKS, CH), jnp.int32),
    ] + [pltpu.VMEM((CH, HIDDEN), jnp.float32) for _ in range(NBUF)]
    + [pltpu.SemaphoreType.DMA for _ in range(2 * NBUF)]
    + [pltpu.VMEM_SHARED((NACC, HIDDEN), jnp.float32)],
    compiler_params=pltpu.CompilerParams(use_tc_tiling_on_sc=False),
)


BLK = 2000
NB1 = N_NODES // BLK

BLK3 = 1000
NB3 = N_NODES // BLK3


def _proj_body(x_ref, w_ref, o_ref):
  o_ref[...] = lax.dot_general(
      x_ref[...], w_ref[...], (((1,), (1,)), ((), ())),
      preferred_element_type=jnp.float32)


_proj = pl.pallas_call(
    _proj_body,
    grid=(NB1,),
    in_specs=[
        pl.BlockSpec((BLK, D_FEAT), lambda i: (i, 0)),
        pl.BlockSpec((HIDDEN, D_FEAT), lambda i: (0, 0)),
    ],
    out_specs=pl.BlockSpec((BLK, HIDDEN), lambda i: (i, 0)),
    out_shape=jax.ShapeDtypeStruct((N_NODES, HIDDEN), jnp.float32),
)


def _stage2_body(y_ref, a_ref, ba_ref, wb_ref, bb_ref, wn_ref, o_ref):
  u = jnp.maximum(y_ref[...] + a_ref[0] + a_ref[1] + ba_ref[...], 0.0)
  h = jnp.maximum(
      lax.dot_general(u, wb_ref[...], (((1,), (1,)), ((), ())),
                      preferred_element_type=jnp.float32) + bb_ref[...], 0.0)
  o_ref[...] = lax.dot_general(
      h, wn_ref[...], (((1,), (1,)), ((), ())),
      preferred_element_type=jnp.float32)


_stage2 = pl.pallas_call(
    _stage2_body,
    grid=(NB1,),
    in_specs=[
        pl.BlockSpec((BLK, HIDDEN), lambda i: (i, 0)),
        pl.BlockSpec((NC, BLK, HIDDEN), lambda i: (0, i, 0)),
        pl.BlockSpec((1, HIDDEN), lambda i: (0, 0)),
        pl.BlockSpec((HIDDEN, HIDDEN), lambda i: (0, 0)),
        pl.BlockSpec((1, HIDDEN), lambda i: (0, 0)),
        pl.BlockSpec((HIDDEN, HIDDEN), lambda i: (0, 0)),
    ],
    out_specs=pl.BlockSpec((BLK, HIDDEN), lambda i: (i, 0)),
    out_shape=jax.ShapeDtypeStruct((N_NODES, HIDDEN), jnp.float32),
)


def _stage3_body(y_ref, a_ref, ba_ref, wb_ref, bb_ref, batch_ref,
                 wfc_ref, bfc_ref, o_ref, sums_ref, cnts_ref):
  i = pl.program_id(0)

  @pl.when(i == 0)
  def _():
    sums_ref[...] = jnp.zeros_like(sums_ref)
    cnts_ref[...] = jnp.zeros_like(cnts_ref)

  u = jnp.maximum(y_ref[...] + a_ref[0] + a_ref[1] + ba_ref[...], 0.0)
  h = jnp.maximum(
      lax.dot_general(u, wb_ref[...], (((1,), (1,)), ((), ())),
                      preferred_element_type=jnp.float32) + bb_ref[...], 0.0)
  bids = batch_ref[0]
  gids = lax.broadcasted_iota(jnp.int32, (N_GRAPHS, BLK3), 0)
  m = (bids == gids).astype(jnp.float32)
  sums_ref[...] += lax.dot_general(
      m, h, (((1,), (0,)), ((), ())), preferred_element_type=jnp.float32)
  cnts_ref[...] += jnp.broadcast_to(
      jnp.sum(m, axis=1, keepdims=True), (N_GRAPHS, HIDDEN))

  @pl.when(i == NB3 - 1)
  def _():
    pooled = sums_ref[...] / jnp.maximum(cnts_ref[...], 1.0)
    o_ref[...] = lax.dot_general(
        pooled, wfc_ref[...], (((1,), (1,)), ((), ())),
        preferred_element_type=jnp.float32) + bfc_ref[...]


_stage3 = pl.pallas_call(
    _stage3_body,
    grid=(NB3,),
    in_specs=[
        pl.BlockSpec((BLK3, HIDDEN), lambda i: (i, 0)),
        pl.BlockSpec((NC, BLK3, HIDDEN), lambda i: (0, i, 0)),
        pl.BlockSpec((1, HIDDEN), lambda i: (0, 0)),
        pl.BlockSpec((HIDDEN, HIDDEN), lambda i: (0, 0)),
        pl.BlockSpec((1, HIDDEN), lambda i: (0, 0)),
        pl.BlockSpec((1, 1, BLK3), lambda i: (i, 0, 0)),
        pl.BlockSpec((N_CLASSES, HIDDEN), lambda i: (0, 0)),
        pl.BlockSpec((1, N_CLASSES), lambda i: (0, 0)),
    ],
    out_specs=pl.BlockSpec((N_GRAPHS, N_CLASSES), lambda i: (0, 0)),
    out_shape=jax.ShapeDtypeStruct((N_GRAPHS, N_CLASSES), jnp.float32),
    scratch_shapes=[
        pltpu.VMEM((N_GRAPHS, HIDDEN), jnp.float32),
        pltpu.VMEM((N_GRAPHS, HIDDEN), jnp.float32),
    ],
)


@jax.jit
def kernel(x, edge_index, batch, W1a, b1a, W1b, b1b, W2a, b2a, W2b, b2b,
           Wfc, bfc):
  ei = edge_index.astype(jnp.int32)
  epad = EPAD - N_EDGES
  sfill = jnp.zeros((epad,), jnp.int32)
  dfill = N_NODES + (jnp.arange(epad, dtype=jnp.int32) % (NACC - N_NODES))
  src = jnp.concatenate([ei[0], sfill]).reshape(NW, CHUNKS, CH)
  dst = jnp.concatenate([ei[1], dfill]).reshape(NW, CHUNKS, CH)
  bpad = batch.astype(jnp.int32).reshape(NB3, 1, BLK3)
  zeros = jnp.zeros((N_NODES, HIDDEN), jnp.float32)

  y1 = _proj(x, W1a)
  a1 = _sc_aggregate(y1, src, dst, zeros)
  y2 = _stage2(y1, a1, b1a.reshape(1, HIDDEN), W1b,
               b1b.reshape(1, HIDDEN), W2a)
  a2 = _sc_aggregate(y2, src, dst, zeros)
  out = _stage3(y2, a2, b2a.reshape(1, HIDDEN), W2b,
                b2b.reshape(1, HIDDEN), bpad, Wfc,
                bfc.reshape(1, N_CLASSES))
  return out


# speedup vs baseline: 2.6081x; 2.6081x over previous
import functools

import jax
import jax.numpy as jnp
from jax import lax
from jax.experimental import pallas as pl
from jax.experimental.pallas import tpu as pltpu
from jax.experimental.pallas import tpu_sc as plsc

N_NODES = 10000
N_EDGES = 320000
D_FEAT = 128
HIDDEN = 64
N_CLASSES = 2
N_GRAPHS = 16

NC = 2
NS = 16
NW = NC * NS

NACC = 10240
CH = 128
EPW = 10240
EPAD = EPW * NW
CHUNKS = EPW // CH
RPW = N_NODES // NS

_SC_MESH = plsc.VectorSubcoreMesh(
    core_axis_name="c", subcore_axis_name="s", num_cores=NC, num_subcores=NS)


NBUF = 8
GROUPS = CHUNKS // NBUF


def _agg_body(y_ref, src_ref, dst_ref, zero_ref, out_ref, *scratch):
  sidx, didx = scratch[0], scratch[1]
  rows = scratch[2:2 + NBUF]
  gsem = scratch[2 + NBUF:2 + 2 * NBUF]
  ssem = scratch[2 + 2 * NBUF:2 + 3 * NBUF]
  acc = scratch[2 + 3 * NBUF]
  c = lax.axis_index("c")
  s = lax.axis_index("s")
  w = c * NS + s
  rbase = s * RPW
  pltpu.sync_copy(src_ref.at[w], sidx)
  pltpu.sync_copy(dst_ref.at[w], didx)
  pltpu.sync_copy(zero_ref.at[pl.ds(rbase, RPW)], acc.at[pl.ds(rbase, RPW)])
  plsc.subcore_barrier()

  for b in range(NBUF):
    pltpu.async_copy(y_ref.at[sidx.at[b]], rows[b], gsem[b])

  def group(j, carry):
    base = j * NBUF
    for b in range(NBUF):
      pltpu.make_async_copy(y_ref.at[sidx.at[base + b]], rows[b],
                            gsem[b]).wait()
      pltpu.async_copy(rows[b], acc.at[didx.at[base + b]], ssem[b], add=True)
    for b in range(NBUF):
      nxt = jnp.minimum(base + NBUF + b, CHUNKS - 1)
      pltpu.make_async_copy(rows[b], acc.at[didx.at[nxt]], ssem[b]).wait()

      @pl.when(j < GROUPS - 1)
      def _():
        pltpu.async_copy(y_ref.at[sidx.at[nxt]], rows[b], gsem[b])
    return carry

  lax.fori_loop(0, GROUPS, group, 0)
  plsc.subcore_barrier()
  pltpu.sync_copy(acc.at[pl.ds(rbase, RPW)], out_ref.at[c, pl.ds(rbase, RPW)])


_sc_aggregate = pl.kernel(
    _agg_body,
    out_type=jax.ShapeDtypeStruct((NC, N_NODES, HIDDEN), jnp.float32),
    mesh=_SC_MESH,
    scratch_types=[
        pltpu.VMEM((CHUNKS, CH), jnp.int32),
        pltpu.VMEM((CHUNKS, CH), jnp.int32),
    ] + [pltpu.VMEM((CH, HIDDEN), jnp.float32) for _ in range(NBUF)]
    + [pltpu.SemaphoreType.DMA for _ in range(2 * NBUF)]
    + [pltpu.VMEM_SHARED((NACC, HIDDEN), jnp.float32)],
    compiler_params=pltpu.CompilerParams(use_tc_tiling_on_sc=False),
)


BLK = 2000
NB1 = N_NODES // BLK

BLK3 = 1000
NB3 = N_NODES // BLK3


def _proj_body(x_ref, w_ref, o_ref):
  o_ref[...] = lax.dot_general(
      x_ref[...], w_ref[...], (((1,), (1,)), ((), ())),
      preferred_element_type=jnp.float32)


_proj = pl.pallas_call(
    _proj_body,
    grid=(NB1,),
    in_specs=[
        pl.BlockSpec((BLK, D_FEAT), lambda i: (i, 0)),
        pl.BlockSpec((HIDDEN, D_FEAT), lambda i: (0, 0)),
    ],
    out_specs=pl.BlockSpec((BLK, HIDDEN), lambda i: (i, 0)),
    out_shape=jax.ShapeDtypeStruct((N_NODES, HIDDEN), jnp.float32),
)


def _stage2_body(y_ref, a_ref, ba_ref, wb_ref, bb_ref, wn_ref, o_ref):
  u = jnp.maximum(y_ref[...] + a_ref[0] + a_ref[1] + ba_ref[...], 0.0)
  h = jnp.maximum(
      lax.dot_general(u, wb_ref[...], (((1,), (1,)), ((), ())),
                      preferred_element_type=jnp.float32) + bb_ref[...], 0.0)
  o_ref[...] = lax.dot_general(
      h, wn_ref[...], (((1,), (1,)), ((), ())),
      preferred_element_type=jnp.float32)


_stage2 = pl.pallas_call(
    _stage2_body,
    grid=(NB1,),
    in_specs=[
        pl.BlockSpec((BLK, HIDDEN), lambda i: (i, 0)),
        pl.BlockSpec((NC, BLK, HIDDEN), lambda i: (0, i, 0)),
        pl.BlockSpec((1, HIDDEN), lambda i: (0, 0)),
        pl.BlockSpec((HIDDEN, HIDDEN), lambda i: (0, 0)),
        pl.BlockSpec((1, HIDDEN), lambda i: (0, 0)),
        pl.BlockSpec((HIDDEN, HIDDEN), lambda i: (0, 0)),
    ],
    out_specs=pl.BlockSpec((BLK, HIDDEN), lambda i: (i, 0)),
    out_shape=jax.ShapeDtypeStruct((N_NODES, HIDDEN), jnp.float32),
)


def _stage3_body(y_ref, a_ref, ba_ref, wb_ref, bb_ref, batch_ref,
                 wfc_ref, bfc_ref, o_ref, sums_ref, cnts_ref):
  i = pl.program_id(0)

  @pl.when(i == 0)
  def _():
    sums_ref[...] = jnp.zeros_like(sums_ref)
    cnts_ref[...] = jnp.zeros_like(cnts_ref)

  u = jnp.maximum(y_ref[...] + a_ref[0] + a_ref[1] + ba_ref[...], 0.0)
  h = jnp.maximum(
      lax.dot_general(u, wb_ref[...], (((1,), (1,)), ((), ())),
                      preferred_element_type=jnp.float32) + bb_ref[...], 0.0)
  bids = batch_ref[0]
  gids = lax.broadcasted_iota(jnp.int32, (N_GRAPHS, BLK3), 0)
  m = (bids == gids).astype(jnp.float32)
  sums_ref[...] += lax.dot_general(
      m, h, (((1,), (0,)), ((), ())), preferred_element_type=jnp.float32)
  cnts_ref[...] += jnp.broadcast_to(
      jnp.sum(m, axis=1, keepdims=True), (N_GRAPHS, HIDDEN))

  @pl.when(i == NB3 - 1)
  def _():
    pooled = sums_ref[...] / jnp.maximum(cnts_ref[...], 1.0)
    o_ref[...] = lax.dot_general(
        pooled, wfc_ref[...], (((1,), (1,)), ((), ())),
        preferred_element_type=jnp.float32) + bfc_ref[...]


_stage3 = pl.pallas_call(
    _stage3_body,
    grid=(NB3,),
    in_specs=[
        pl.BlockSpec((BLK3, HIDDEN), lambda i: (i, 0)),
        pl.BlockSpec((NC, BLK3, HIDDEN), lambda i: (0, i, 0)),
        pl.BlockSpec((1, HIDDEN), lambda i: (0, 0)),
        pl.BlockSpec((HIDDEN, HIDDEN), lambda i: (0, 0)),
        pl.BlockSpec((1, HIDDEN), lambda i: (0, 0)),
        pl.BlockSpec((1, 1, BLK3), lambda i: (i, 0, 0)),
        pl.BlockSpec((N_CLASSES, HIDDEN), lambda i: (0, 0)),
        pl.BlockSpec((1, N_CLASSES), lambda i: (0, 0)),
    ],
    out_specs=pl.BlockSpec((N_GRAPHS, N_CLASSES), lambda i: (0, 0)),
    out_shape=jax.ShapeDtypeStruct((N_GRAPHS, N_CLASSES), jnp.float32),
    scratch_shapes=[
        pltpu.VMEM((N_GRAPHS, HIDDEN), jnp.float32),
        pltpu.VMEM((N_GRAPHS, HIDDEN), jnp.float32),
    ],
)


@jax.jit
def kernel(x, edge_index, batch, W1a, b1a, W1b, b1b, W2a, b2a, W2b, b2b,
           Wfc, bfc):
  ei = edge_index.astype(jnp.int32)
  epad = EPAD - N_EDGES
  sfill = jnp.arange(epad, dtype=jnp.int32) % (NACC - N_NODES)
  dfill = N_NODES + (jnp.arange(epad, dtype=jnp.int32) % (NACC - N_NODES))
  src = jnp.concatenate([ei[0], sfill]).reshape(NW, CHUNKS, CH)
  dst = jnp.concatenate([ei[1], dfill]).reshape(NW, CHUNKS, CH)
  bpad = batch.astype(jnp.int32).reshape(NB3, 1, BLK3)
  zeros = jnp.zeros((N_NODES, HIDDEN), jnp.float32)

  y1 = _proj(x, W1a)
  a1 = _sc_aggregate(y1, src, dst, zeros)
  y2 = _stage2(y1, a1, b1a.reshape(1, HIDDEN), W1b,
               b1b.reshape(1, HIDDEN), W2a)
  a2 = _sc_aggregate(y2, src, dst, zeros)
  out = _stage3(y2, a2, b2a.reshape(1, HIDDEN), W2b,
                b2b.reshape(1, HIDDEN), bpad, Wfc,
                bfc.reshape(1, N_CLASSES))
  return out


# speedup vs baseline: 2.6847x; 1.0294x over previous
import jax
import jax.numpy as jnp
from jax import lax
from jax.experimental import pallas as pl
from jax.experimental.pallas import tpu as pltpu
from jax.experimental.pallas import tpu_sc as plsc

N_NODES = 10000
N_EDGES = 320000
D_FEAT = 128
HIDDEN = 64
N_CLASSES = 2
N_GRAPHS = 16

NC = 2
NS = 16
NW = NC * NS

NACC = 10240
CH = 128
EPW = 10240
EPAD = EPW * NW
CHUNKS = EPW // CH
RPW = N_NODES // NS

NBUF = 8
GROUPS = CHUNKS // NBUF

_SC_MESH = plsc.VectorSubcoreMesh(
    core_axis_name="c", subcore_axis_name="s", num_cores=NC, num_subcores=NS)


def _agg_body(y_ref, src_ref, dst_ref, zero_ref, out_ref, *scratch):
  sidx, didx = scratch[0], scratch[1]
  rows = scratch[2:2 + NBUF]
  gsem = scratch[2 + NBUF:2 + 2 * NBUF]
  ssem = scratch[2 + 2 * NBUF:2 + 3 * NBUF]
  acc = scratch[2 + 3 * NBUF]
  c = lax.axis_index("c")
  s = lax.axis_index("s")
  w = c * NS + s
  rbase = s * RPW
  pltpu.sync_copy(src_ref.at[pl.ds(w * EPW, EPW)], sidx)
  pltpu.sync_copy(dst_ref.at[w], didx)
  pltpu.sync_copy(zero_ref.at[pl.ds(rbase, RPW)], acc.at[pl.ds(rbase, RPW)])
  plsc.subcore_barrier()

  for b in range(NBUF):
    pltpu.async_copy(y_ref.at[sidx.at[pl.ds(b * CH, CH)]], rows[b], gsem[b])

  def group(j, carry):
    base = j * NBUF
    for b in range(NBUF):
      pltpu.make_async_copy(
          y_ref.at[sidx.at[pl.ds((base + b) * CH, CH)]], rows[b],
          gsem[b]).wait()
      pltpu.async_copy(rows[b], acc.at[didx.at[base + b]], ssem[b], add=True)
    for b in range(NBUF):
      nxt = jnp.minimum(base + NBUF + b, CHUNKS - 1)
      pltpu.make_async_copy(rows[b], acc.at[didx.at[nxt]], ssem[b]).wait()

      @pl.when(j < GROUPS - 1)
      def _():
        pltpu.async_copy(y_ref.at[sidx.at[pl.ds(nxt * CH, CH)]], rows[b],
                         gsem[b])
    return carry

  lax.fori_loop(0, GROUPS, group, 0)
  plsc.subcore_barrier()
  pltpu.sync_copy(acc.at[pl.ds(rbase, RPW)], out_ref.at[c, pl.ds(rbase, RPW)])


_sc_aggregate = pl.kernel(
    _agg_body,
    out_type=jax.ShapeDtypeStruct((NC, N_NODES, HIDDEN), jnp.float32),
    mesh=_SC_MESH,
    scratch_types=[
        pltpu.VMEM((EPW,), jnp.int32),
        pltpu.VMEM((CHUNKS, CH), jnp.int32),
    ] + [pltpu.VMEM((CH, HIDDEN), jnp.float32) for _ in range(NBUF)]
    + [pltpu.SemaphoreType.DMA for _ in range(2 * NBUF)]
    + [pltpu.VMEM_SHARED((NACC, HIDDEN), jnp.float32)],
    compiler_params=pltpu.CompilerParams(use_tc_tiling_on_sc=False),
)


def _proj_body(x_ref, w_ref, o_ref):
  o_ref[...] = lax.dot_general(
      x_ref[...], w_ref[...], (((1,), (1,)), ((), ())),
      preferred_element_type=jnp.float32)


_proj = pl.pallas_call(
    _proj_body,
    out_shape=jax.ShapeDtypeStruct((N_NODES, HIDDEN), jnp.float32),
)


def _stage2_body(y_ref, a_ref, ba_ref, wb_ref, bb_ref, wn_ref, o_ref):
  u = jnp.maximum(y_ref[...] + a_ref[0] + a_ref[1] + ba_ref[...], 0.0)
  h = jnp.maximum(
      lax.dot_general(u, wb_ref[...], (((1,), (1,)), ((), ())),
                      preferred_element_type=jnp.float32) + bb_ref[...], 0.0)
  o_ref[...] = lax.dot_general(
      h, wn_ref[...], (((1,), (1,)), ((), ())),
      preferred_element_type=jnp.float32)


_stage2 = pl.pallas_call(
    _stage2_body,
    out_shape=jax.ShapeDtypeStruct((N_NODES, HIDDEN), jnp.float32),
)


def _stage3_body(y_ref, a_ref, ba_ref, wb_ref, bb_ref, batch_ref,
                 wfc_ref, bfc_ref, o_ref):
  u = jnp.maximum(y_ref[...] + a_ref[0] + a_ref[1] + ba_ref[...], 0.0)
  h = jnp.maximum(
      lax.dot_general(u, wb_ref[...], (((1,), (1,)), ((), ())),
                      preferred_element_type=jnp.float32) + bb_ref[...], 0.0)
  bids = batch_ref[...]
  gids = lax.broadcasted_iota(jnp.int32, (N_GRAPHS, N_NODES), 0)
  m = (bids == gids).astype(jnp.float32)
  sums = lax.dot_general(
      m, h, (((1,), (0,)), ((), ())), preferred_element_type=jnp.float32)
  cnts = jnp.sum(m, axis=1, keepdims=True)
  pooled = sums / jnp.maximum(cnts, 1.0)
  o_ref[...] = lax.dot_general(
      pooled, wfc_ref[...], (((1,), (1,)), ((), ())),
      preferred_element_type=jnp.float32) + bfc_ref[...]


_stage3 = pl.pallas_call(
    _stage3_body,
    out_shape=jax.ShapeDtypeStruct((N_GRAPHS, N_CLASSES), jnp.float32),
)


@jax.jit
def kernel(x, edge_index, batch, W1a, b1a, W1b, b1b, W2a, b2a, W2b, b2b,
           Wfc, bfc):
  ei = edge_index.astype(jnp.int32)
  epad = EPAD - N_EDGES
  fill = jnp.arange(epad, dtype=jnp.int32) % (NACC - N_NODES)
  src = jnp.concatenate([ei[0], fill])
  dst = jnp.concatenate([ei[1], N_NODES + fill]).reshape(NW, CHUNKS, CH)
  batch2d = batch.astype(jnp.int32).reshape(1, N_NODES)
  zeros = jnp.zeros((N_NODES, HIDDEN), jnp.float32)

  y1 = _proj(x, W1a)
  a1 = _sc_aggregate(y1, src, dst, zeros)
  y2 = _stage2(y1, a1, b1a.reshape(1, HIDDEN), W1b,
               b1b.reshape(1, HIDDEN), W2a)
  a2 = _sc_aggregate(y2, src, dst, zeros)
  out = _stage3(y2, a2, b2a.reshape(1, HIDDEN), W2b,
                b2b.reshape(1, HIDDEN), batch2d, Wfc,
                bfc.reshape(1, N_CLASSES))
  return out


# speedup vs baseline: 2.6915x; 1.0025x over previous
import jax
import jax.numpy as jnp
from jax import lax
from jax.experimental import pallas as pl
from jax.experimental.pallas import tpu as pltpu
from jax.experimental.pallas import tpu_sc as plsc

N_NODES = 10000
N_EDGES = 320000
D_FEAT = 128
HIDDEN = 64
N_CLASSES = 2
N_GRAPHS = 16

NC = 2
NS = 16
NW = NC * NS

NACC = 10240
CH = 128
EPW = 10240
EPAD = EPW * NW
CHUNKS = EPW // CH
RPW = N_NODES // NS

NBUF = 8
GROUPS = CHUNKS // NBUF

_SC_MESH = plsc.VectorSubcoreMesh(
    core_axis_name="c", subcore_axis_name="s", num_cores=NC, num_subcores=NS)


def _agg_body(y_ref, src_ref, dst_ref, zero_ref, out_ref, *scratch):
  sidx, didx = scratch[0], scratch[1]
  rows = scratch[2:2 + NBUF]
  gsem = scratch[2 + NBUF:2 + 2 * NBUF]
  ssem = scratch[2 + 2 * NBUF:2 + 3 * NBUF]
  acc = scratch[2 + 3 * NBUF]
  c = lax.axis_index("c")
  s = lax.axis_index("s")
  w = c * NS + s
  rbase = s * RPW
  pltpu.sync_copy(src_ref.at[pl.ds(w * EPW, EPW)], sidx)
  pltpu.sync_copy(dst_ref.at[pl.ds(w * EPW, EPW)], didx)
  pltpu.sync_copy(zero_ref.at[pl.ds(rbase, RPW)], acc.at[pl.ds(rbase, RPW)])
  plsc.subcore_barrier()

  for b in range(NBUF):
    pltpu.async_copy(y_ref.at[sidx.at[pl.ds(b * CH, CH)]], rows[b], gsem[b])

  def group(j, carry):
    base = j * NBUF
    for b in range(NBUF):
      pltpu.make_async_copy(
          y_ref.at[sidx.at[pl.ds((base + b) * CH, CH)]], rows[b],
          gsem[b]).wait()
      pltpu.async_copy(rows[b], acc.at[didx.at[pl.ds((base + b) * CH, CH)]], ssem[b], add=True)
    for b in range(NBUF):
      nxt = jnp.minimum(base + NBUF + b, CHUNKS - 1)
      pltpu.make_async_copy(rows[b], acc.at[didx.at[pl.ds(nxt * CH, CH)]], ssem[b]).wait()

      @pl.when(j < GROUPS - 1)
      def _():
        pltpu.async_copy(y_ref.at[sidx.at[pl.ds(nxt * CH, CH)]], rows[b],
                         gsem[b])
    return carry

  lax.fori_loop(0, GROUPS, group, 0)
  plsc.subcore_barrier()
  pltpu.sync_copy(acc.at[pl.ds(rbase, RPW)], out_ref.at[c, pl.ds(rbase, RPW)])


_sc_aggregate = pl.kernel(
    _agg_body,
    out_type=jax.ShapeDtypeStruct((NC, N_NODES, HIDDEN), jnp.float32),
    mesh=_SC_MESH,
    scratch_types=[
        pltpu.VMEM((EPW,), jnp.int32),
        pltpu.VMEM((EPW,), jnp.int32),
    ] + [pltpu.VMEM((CH, HIDDEN), jnp.float32) for _ in range(NBUF)]
    + [pltpu.SemaphoreType.DMA for _ in range(2 * NBUF)]
    + [pltpu.VMEM_SHARED((NACC, HIDDEN), jnp.float32)],
    compiler_params=pltpu.CompilerParams(use_tc_tiling_on_sc=False),
)


def _proj_body(x_ref, w_ref, o_ref):
  o_ref[...] = lax.dot_general(
      x_ref[...], w_ref[...], (((1,), (1,)), ((), ())),
      preferred_element_type=jnp.float32)


_proj = pl.pallas_call(
    _proj_body,
    out_shape=jax.ShapeDtypeStruct((N_NODES, HIDDEN), jnp.float32),
)


def _stage2_body(y_ref, a_ref, ba_ref, wb_ref, bb_ref, wn_ref, o_ref):
  u = jnp.maximum(y_ref[...] + a_ref[0] + a_ref[1] + ba_ref[...], 0.0)
  h = jnp.maximum(
      lax.dot_general(u, wb_ref[...], (((1,), (1,)), ((), ())),
                      preferred_element_type=jnp.float32) + bb_ref[...], 0.0)
  o_ref[...] = lax.dot_general(
      h, wn_ref[...], (((1,), (1,)), ((), ())),
      preferred_element_type=jnp.float32)


_stage2 = pl.pallas_call(
    _stage2_body,
    out_shape=jax.ShapeDtypeStruct((N_NODES, HIDDEN), jnp.float32),
)


def _stage3_body(y_ref, a_ref, ba_ref, wb_ref, bb_ref, batch_ref,
                 wfc_ref, bfc_ref, o_ref):
  u = jnp.maximum(y_ref[...] + a_ref[0] + a_ref[1] + ba_ref[...], 0.0)
  h = jnp.maximum(
      lax.dot_general(u, wb_ref[...], (((1,), (1,)), ((), ())),
                      preferred_element_type=jnp.float32) + bb_ref[...], 0.0)
  bids = batch_ref[...]
  gids = lax.broadcasted_iota(jnp.int32, (N_GRAPHS, N_NODES), 0)
  m = (bids == gids).astype(jnp.float32)
  sums = lax.dot_general(
      m, h, (((1,), (0,)), ((), ())), preferred_element_type=jnp.float32)
  cnts = jnp.sum(m, axis=1, keepdims=True)
  pooled = sums / jnp.maximum(cnts, 1.0)
  o_ref[...] = lax.dot_general(
      pooled, wfc_ref[...], (((1,), (1,)), ((), ())),
      preferred_element_type=jnp.float32) + bfc_ref[...]


_stage3 = pl.pallas_call(
    _stage3_body,
    out_shape=jax.ShapeDtypeStruct((N_GRAPHS, N_CLASSES), jnp.float32),
)


@jax.jit
def kernel(x, edge_index, batch, W1a, b1a, W1b, b1b, W2a, b2a, W2b, b2b,
           Wfc, bfc):
  ei = edge_index.astype(jnp.int32)
  epad = EPAD - N_EDGES
  fill = jnp.arange(epad, dtype=jnp.int32) % (NACC - N_NODES)
  src = jnp.concatenate([ei[0], fill])
  dst = jnp.concatenate([ei[1], N_NODES + fill])
  batch2d = batch.astype(jnp.int32).reshape(1, N_NODES)
  zeros = jnp.zeros((N_NODES, HIDDEN), jnp.float32)

  y1 = _proj(x, W1a)
  a1 = _sc_aggregate(y1, src, dst, zeros)
  y2 = _stage2(y1, a1, b1a.reshape(1, HIDDEN), W1b,
               b1b.reshape(1, HIDDEN), W2a)
  a2 = _sc_aggregate(y2, src, dst, zeros)
  out = _stage3(y2, a2, b2a.reshape(1, HIDDEN), W2b,
                b2b.reshape(1, HIDDEN), batch2d, Wfc,
                bfc.reshape(1, N_CLASSES))
  return out


# speedup vs baseline: 2.9558x; 1.0982x over previous
import jax
import jax.numpy as jnp
from jax import lax
from jax.experimental import pallas as pl
from jax.experimental.pallas import tpu as pltpu
from jax.experimental.pallas import tpu_sc as plsc

N_NODES = 10000
N_EDGES = 320000
D_FEAT = 128
HIDDEN = 64
N_CLASSES = 2
N_GRAPHS = 16

NC = 2
NS = 16
NW = NC * NS

NACC = 10240
CH = 128
EPW = 10240
EPAD = EPW * NW
CHUNKS = EPW // CH
RPW = N_NODES // NS

NBUF = 8
GROUPS = CHUNKS // NBUF

_SC_MESH = plsc.VectorSubcoreMesh(
    core_axis_name="c", subcore_axis_name="s", num_cores=NC, num_subcores=NS)


def _agg_body(y_ref, src_ref, dst_ref, zero_ref, out_ref, *scratch):
  sidx, didx = scratch[0], scratch[1]
  rows = scratch[2:2 + NBUF]
  gsem = scratch[2 + NBUF:2 + 2 * NBUF]
  ssem = scratch[2 + 2 * NBUF:2 + 3 * NBUF]
  acc = scratch[2 + 3 * NBUF]
  c = lax.axis_index("c")
  s = lax.axis_index("s")
  w = c * NS + s
  rbase = s * RPW
  pltpu.sync_copy(src_ref.at[pl.ds(w * EPW, EPW)], sidx)
  pltpu.sync_copy(dst_ref.at[pl.ds(w * EPW, EPW)], didx)
  pltpu.sync_copy(zero_ref.at[pl.ds(rbase, RPW)], acc.at[pl.ds(rbase, RPW)])
  plsc.subcore_barrier()

  for b in range(NBUF):
    pltpu.async_copy(y_ref.at[sidx.at[pl.ds(b * CH, CH)]], rows[b], gsem[b])

  def group(j, carry):
    base = j * NBUF
    for b in range(NBUF):
      pltpu.make_async_copy(
          y_ref.at[sidx.at[pl.ds((base + b) * CH, CH)]], rows[b],
          gsem[b]).wait()
      pltpu.async_copy(rows[b], acc.at[didx.at[pl.ds((base + b) * CH, CH)]], ssem[b], add=True)
    for b in range(NBUF):
      nxt = jnp.minimum(base + NBUF + b, CHUNKS - 1)
      pltpu.make_async_copy(rows[b], acc.at[didx.at[pl.ds(nxt * CH, CH)]], ssem[b]).wait()

      @pl.when(j < GROUPS - 1)
      def _():
        pltpu.async_copy(y_ref.at[sidx.at[pl.ds(nxt * CH, CH)]], rows[b],
                         gsem[b])
    return carry

  lax.fori_loop(0, GROUPS, group, 0)
  plsc.subcore_barrier()
  pltpu.sync_copy(acc.at[pl.ds(rbase, RPW)],
                  out_ref.at[c, pl.ds(rbase, RPW), pl.ds(0, HIDDEN)])


_sc_aggregate = pl.kernel(
    _agg_body,
    out_type=jax.ShapeDtypeStruct((NC, N_NODES, 2 * HIDDEN), jnp.float32),
    mesh=_SC_MESH,
    scratch_types=[
        pltpu.VMEM((EPW,), jnp.int32),
        pltpu.VMEM((EPW,), jnp.int32),
    ] + [pltpu.VMEM((CH, HIDDEN), jnp.float32) for _ in range(NBUF)]
    + [pltpu.SemaphoreType.DMA for _ in range(2 * NBUF)]
    + [pltpu.VMEM_SHARED((NACC, HIDDEN), jnp.float32)],
    compiler_params=pltpu.CompilerParams(use_tc_tiling_on_sc=False),
)


def _proj_body(x_ref, w_ref, o_ref):
  o_ref[...] = lax.dot_general(
      x_ref[...], w_ref[...], (((1,), (1,)), ((), ())),
      preferred_element_type=jnp.float32)


_proj = pl.pallas_call(
    _proj_body,
    out_shape=jax.ShapeDtypeStruct((N_NODES, HIDDEN), jnp.float32),
)


def _stage2_body(y_ref, a_ref, ba_ref, wb_ref, bb_ref, wn_ref, o_ref):
  a0 = a_ref[0, :, 0:HIDDEN]
  a1 = a_ref[1, :, 0:HIDDEN]
  u = jnp.maximum(y_ref[...] + a0 + a1 + ba_ref[...], 0.0)
  h = jnp.maximum(
      lax.dot_general(u, wb_ref[...], (((1,), (1,)), ((), ())),
                      preferred_element_type=jnp.float32) + bb_ref[...], 0.0)
  o_ref[...] = lax.dot_general(
      h, wn_ref[...], (((1,), (1,)), ((), ())),
      preferred_element_type=jnp.float32)


_stage2 = pl.pallas_call(
    _stage2_body,
    out_shape=jax.ShapeDtypeStruct((N_NODES, HIDDEN), jnp.float32),
)


def _stage3_body(y_ref, a_ref, ba_ref, wb_ref, bb_ref, batch_ref,
                 wfc_ref, bfc_ref, o_ref):
  a0 = a_ref[0, :, 0:HIDDEN]
  a1 = a_ref[1, :, 0:HIDDEN]
  u = jnp.maximum(y_ref[...] + a0 + a1 + ba_ref[...], 0.0)
  h = jnp.maximum(
      lax.dot_general(u, wb_ref[...], (((1,), (1,)), ((), ())),
                      preferred_element_type=jnp.float32) + bb_ref[...], 0.0)
  bids = batch_ref[...]
  gids = lax.broadcasted_iota(jnp.int32, (N_GRAPHS, N_NODES), 0)
  m = (bids == gids).astype(jnp.float32)
  sums = lax.dot_general(
      m, h, (((1,), (0,)), ((), ())), preferred_element_type=jnp.float32)
  cnts = jnp.sum(m, axis=1, keepdims=True)
  pooled = sums / jnp.maximum(cnts, 1.0)
  o_ref[...] = lax.dot_general(
      pooled, wfc_ref[...], (((1,), (1,)), ((), ())),
      preferred_element_type=jnp.float32) + bfc_ref[...]


_stage3 = pl.pallas_call(
    _stage3_body,
    out_shape=jax.ShapeDtypeStruct((N_GRAPHS, N_CLASSES), jnp.float32),
)


@jax.jit
def kernel(x, edge_index, batch, W1a, b1a, W1b, b1b, W2a, b2a, W2b, b2b,
           Wfc, bfc):
  ei = edge_index
  epad = EPAD - N_EDGES
  fill = jnp.arange(epad, dtype=jnp.int32) % (NACC - N_NODES)
  src = jnp.concatenate([ei[0], fill])
  dst = jnp.concatenate([ei[1], N_NODES + fill])
  batch2d = batch.astype(jnp.int32).reshape(1, N_NODES)
  zeros = jnp.zeros((N_NODES, HIDDEN), jnp.float32)

  y1 = _proj(x, W1a)
  a1 = _sc_aggregate(y1, src, dst, zeros)
  y2 = _stage2(y1, a1, b1a.reshape(1, HIDDEN), W1b,
               b1b.reshape(1, HIDDEN), W2a)
  a2 = _sc_aggregate(y2, src, dst, zeros)
  out = _stage3(y2, a2, b2a.reshape(1, HIDDEN), W2b,
                b2b.reshape(1, HIDDEN), batch2d, Wfc,
                bfc.reshape(1, N_CLASSES))
  return out
